# Initial kernel scaffold; baseline (speedup 1.0000x reference)
import jax
import jax.numpy as jnp
from jax.experimental import pallas as pl


def kernel(x, edge_index, W1, as1, ad1, b1, g1, be1, W2, as2, ad2, b2, g2, be2, W3, as3, ad3, b3, Wc, bc):
    raise NotImplementedError("write your pallas kernel here")


# speedup vs baseline: 1.0001x; 1.0001x over previous
import jax
import jax.numpy as jnp
from jax.experimental import pallas as pl


def _gat_conv(x, src, dst, W, a_src, a_dst, bias, H, C, concat):
    n = x.shape[0]
    h = (x @ W).reshape(n, H, C)
    e_src = jnp.sum(h * a_src[None, :, :], axis=-1)
    e_dst = jnp.sum(h * a_dst[None, :, :], axis=-1)
    alpha = e_src[src] + e_dst[dst]
    alpha = jax.nn.leaky_relu(alpha, negative_slope=0.2)
    amax = jax.ops.segment_max(alpha, dst, num_segments=n)
    amax = jnp.where(jnp.isfinite(amax), amax, 0.0)
    ex = jnp.exp(alpha - amax[dst])
    denom = jax.ops.segment_sum(ex, dst, num_segments=n)
    att = ex / (denom[dst] + 1e-16)
    msg = h[src] * att[:, :, None]
    out = jax.ops.segment_sum(msg, dst, num_segments=n)
    if concat:
        out = out.reshape(n, H * C)
    else:
        out = out.mean(axis=1)
    return out + bias


def _bn(x, g, b):
    mean = jnp.mean(x, axis=0)
    var = jnp.var(x, axis=0)
    return (x - mean) / jnp.sqrt(var + 1e-5) * g + b


def _mm_kernel(a_ref, b_ref, o_ref):
    o_ref[...] = a_ref[...] @ b_ref[...]


def _pallas_mm(a, b):
    return pl.pallas_call(
        _mm_kernel,
        out_shape=jax.ShapeDtypeStruct((a.shape[0], b.shape[1]), a.dtype),
    )(a, b)


def kernel(x, edge_index, W1, as1, ad1, b1, g1, be1, W2, as2, ad2, b2, g2, be2, W3, as3, ad3, b3, Wc, bc):
    n = x.shape[0]
    loop = jnp.arange(n, dtype=edge_index.dtype)
    src = jnp.concatenate([edge_index[0], loop])
    dst = jnp.concatenate([edge_index[1], loop])
    h = _gat_conv(x, src, dst, W1, as1, ad1, b1, 4, 256, True)
    h = jax.nn.elu(_bn(h, g1, be1))
    h = _gat_conv(h, src, dst, W2, as2, ad2, b2, 4, 128, True)
    h = jax.nn.elu(_bn(h, g2, be2))
    emb = _gat_conv(h, src, dst, W3, as3, ad3, b3, 1, 64, False)
    logits = emb @ Wc + bc
    return (logits, emb)


# speedup vs baseline: 2.3339x; 2.3338x over previous
import functools

import jax
import jax.numpy as jnp
from jax import lax
from jax.experimental import pallas as pl
from jax.experimental.pallas import tpu as pltpu
from jax.experimental.pallas import tpu_sc as plsc

N = 10000
E_RAW = 320000
E_TOT = E_RAW + N
EPAD = 335872
B = 256
HALF = 5120
HROWS = 5248
RPT = 328
RPT2 = 320
OUTR = 2 * HALF


def _mm_body(x_ref, w_ref, o_ref):
    o_ref[...] = jnp.dot(x_ref[...], w_ref[...],
                         preferred_element_type=jnp.float32)


def _pallas_mm(x, w, rows_blk=400):
    m, k = x.shape
    f = w.shape[1]
    return pl.pallas_call(
        _mm_body,
        grid=(m // rows_blk,),
        in_specs=[pl.BlockSpec((rows_blk, k), lambda i: (i, 0)),
                  pl.BlockSpec((k, f), lambda i: (0, 0))],
        out_specs=pl.BlockSpec((rows_blk, f), lambda i: (i, 0)),
        out_shape=jax.ShapeDtypeStruct((m, f), jnp.float32),
    )(x, w)


def _make_agg(nchunk, ch_per_head):
    epw = EPAD // 16
    nbatch = epw // B
    mesh = plsc.VectorSubcoreMesh(core_axis_name="c", subcore_axis_name="s")

    @functools.partial(
        pl.kernel, mesh=mesh,
        out_type=jax.ShapeDtypeStruct((nchunk * OUTR, 128), jnp.float32),
        scratch_types=[
            pltpu.VMEM((B,), jnp.int32),
            pltpu.VMEM((B,), jnp.int32),
            pltpu.VMEM((B,), jnp.float32),
            pltpu.VMEM((B, 128), jnp.float32),
            pltpu.VMEM((RPT, 128), jnp.float32),
            pltpu.VMEM_SHARED((HROWS, 128), jnp.float32),
            pltpu.SemaphoreType.DMA,
        ],
    )
    def k(h3, srcp, dstp, attf, out_hbm, srcv, dstv, attv, rowsv, zv,
          out_sh, sem):
        cid = lax.axis_index("c")
        sid = lax.axis_index("s")
        lo16 = jnp.full((16,), 0, jnp.int32) + cid * HALF

        def zero_body(i, _):
            for j in range(8):
                zv[i, pl.ds(j * 16, 16)] = jnp.zeros((16,), jnp.float32)
            return 0

        lax.fori_loop(0, RPT, zero_body, 0)

        for c in range(nchunk):
            head = c // ch_per_head
            base_e = sid * epw
            pltpu.sync_copy(zv, out_sh.at[pl.ds(sid * RPT, RPT)])
            plsc.subcore_barrier()

            def batch_body(b, _):
                base = base_e + b * B
                pltpu.sync_copy(srcp.at[pl.ds(base, B)], srcv)
                pltpu.sync_copy(dstp.at[pl.ds(base, B)], dstv)
                pltpu.sync_copy(attf.at[pl.ds(head * EPAD + base, B)], attv)
                coff = jnp.full((16,), c * N, jnp.int32)

                def idx_body(i, _):
                    srcv[pl.ds(i * 16, 16)] = srcv[pl.ds(i * 16, 16)] + coff
                    dv = dstv[pl.ds(i * 16, 16)] - lo16
                    ok = (dv >= 0) & (dv < HALF)
                    dstv[pl.ds(i * 16, 16)] = jnp.where(
                        ok, dv, jnp.full((16,), HALF, jnp.int32))
                    return 0

                lax.fori_loop(0, B // 16, idx_body, 0)
                pltpu.async_copy(h3.at[srcv], rowsv, sem).wait()

                def mul_body(g, _):
                    av = attv[pl.ds(g * 16, 16)]
                    for lane in range(16):
                        s = av[lane]
                        r = g * 16 + lane
                        for j in range(8):
                            rowsv[r, pl.ds(j * 16, 16)] = (
                                rowsv[r, pl.ds(j * 16, 16)] * s)
                    return 0

                lax.fori_loop(0, B // 16, mul_body, 0)
                pltpu.sync_copy(rowsv, out_sh.at[dstv], add=True)
                return 0

            lax.fori_loop(0, nbatch, batch_body, 0)
            plsc.subcore_barrier()
            pltpu.sync_copy(
                out_sh.at[pl.ds(sid * RPT2, RPT2)],
                out_hbm.at[pl.ds(c * OUTR + cid * HALF + sid * RPT2, RPT2)])
            plsc.subcore_barrier()

    return k


_agg_l1 = _make_agg(8, 2)
_agg_l2 = _make_agg(4, 1)
_agg_l3 = _make_agg(1, 1)


def _gat_layer(x, srcp, dstp, src, dst, W, a_src, a_dst, bias, H, C,
               agg, nchunk, w_keep):
    n = x.shape[0]
    h = _pallas_mm(x, W)
    hh = h.reshape(n, H, C)
    e_src = jnp.sum(hh * a_src[None, :, :], axis=-1)
    e_dst = jnp.sum(hh * a_dst[None, :, :], axis=-1)
    alpha = e_src[src] + e_dst[dst]
    alpha = jax.nn.leaky_relu(alpha, negative_slope=0.2)
    amax = jax.ops.segment_max(alpha, dst, num_segments=n)
    amax = jnp.where(jnp.isfinite(amax), amax, 0.0)
    ex = jnp.exp(alpha - amax[dst])
    denom = jax.ops.segment_sum(ex, dst, num_segments=n)
    att = ex / (denom[dst] + 1e-16)
    attf = jnp.zeros((H, EPAD), jnp.float32).at[:, :E_TOT].set(att.T).ravel()
    if w_keep == 128:
        h3 = h.reshape(n, nchunk, 128).transpose(1, 0, 2).reshape(
            nchunk * n, 128)
    else:
        h3 = jnp.pad(h, ((0, 0), (0, 128 - w_keep)))
    out3 = agg(h3, srcp, dstp, attf)
    out3 = out3.reshape(nchunk, OUTR, 128)[:, :n, :w_keep]
    out = out3.transpose(1, 0, 2).reshape(n, nchunk * w_keep)
    return out + bias


def _bn(x, g, b):
    mean = jnp.mean(x, axis=0)
    var = jnp.var(x, axis=0)
    return (x - mean) / jnp.sqrt(var + 1e-5) * g + b


def kernel(x, edge_index, W1, as1, ad1, b1, g1, be1, W2, as2, ad2, b2,
           g2, be2, W3, as3, ad3, b3, Wc, bc):
    n = x.shape[0]
    loop = jnp.arange(n, dtype=edge_index.dtype)
    src = jnp.concatenate([edge_index[0], loop])
    dst = jnp.concatenate([edge_index[1], loop])
    pad = EPAD - E_TOT
    srcp = jnp.concatenate([src, jnp.zeros((pad,), jnp.int32)])
    dstp = jnp.concatenate([dst, jnp.full((pad,), n, jnp.int32)])

    h = _gat_layer(x, srcp, dstp, src, dst, W1, as1, ad1, b1, 4, 256,
                   _agg_l1, 8, 128)
    h = jax.nn.elu(_bn(h, g1, be1))
    h = _gat_layer(h, srcp, dstp, src, dst, W2, as2, ad2, b2, 4, 128,
                   _agg_l2, 4, 128)
    h = jax.nn.elu(_bn(h, g2, be2))
    emb = _gat_layer(h, srcp, dstp, src, dst, W3, as3, ad3, b3, 1, 64,
                     _agg_l3, 1, 64)
    logits = _pallas_mm(emb, Wc, rows_blk=400) + bc
    return (logits, emb)
